# Initial kernel scaffold; baseline (speedup 1.0000x reference)
import jax
import jax.numpy as jnp
from jax.experimental import pallas as pl


def kernel(x, edge_index, Wz, bz, Lz, lbz, Wr, br, Lr, lbr, Wh, bh, Lh, lbh, att, Wlin, blin):
    raise NotImplementedError("write your pallas kernel here")


# speedup vs baseline: 162.9532x; 162.9532x over previous
import functools

import jax
import jax.numpy as jnp
from jax import lax
from jax.experimental import pallas as pl
from jax.experimental.pallas import tpu as pltpu
from jax.experimental.pallas import tpu_sc as plsc

NCORE = 2
NSUB = 16
CH = 128
F = 24
HID = 32
P = 12
BN = 512


def _sc_degree(dst3, zrow, np_, nch):
    per = np_ // NSUB
    mesh = plsc.VectorSubcoreMesh(core_axis_name="c", subcore_axis_name="s")

    @functools.partial(
        pl.kernel,
        out_type=jax.ShapeDtypeStruct((NCORE * np_,), jnp.float32),
        mesh=mesh,
        scratch_types=[
            pltpu.VMEM((nch, CH), jnp.int32),
            pltpu.VMEM((CH,), jnp.float32),
            pltpu.VMEM_SHARED((np_,), jnp.float32),
        ],
    )
    def k(dst_h, z_h, out_h, idx_v, ones_v, deg_s):
        cid = lax.axis_index("c")
        sid = lax.axis_index("s")
        wid = cid * NSUB + sid
        pltpu.sync_copy(z_h, deg_s.at[pl.ds(sid * per, per)])
        pltpu.sync_copy(dst_h.at[wid], idx_v)
        for i in range(CH // 16):
            ones_v[pl.ds(i * 16, 16)] = jnp.ones((16,), jnp.float32)
        plsc.subcore_barrier()

        def body(j, carry):
            pltpu.sync_copy(ones_v, deg_s.at[idx_v.at[j]], add=True)
            return carry

        lax.fori_loop(0, nch, body, 0)
        plsc.subcore_barrier()
        off = pl.multiple_of(wid * per, per)
        pltpu.sync_copy(deg_s.at[pl.ds(sid * per, per)],
                        out_h.at[pl.ds(off, per)])

    return k(dst3, zrow)


def _sc_scatter(xs, src3, dst3, ztile, np_, nch):
    per = np_ // NSUB
    mesh = plsc.VectorSubcoreMesh(core_axis_name="c", subcore_axis_name="s")

    @functools.partial(
        pl.kernel,
        out_type=jax.ShapeDtypeStruct((NCORE * np_, F), jnp.float32),
        mesh=mesh,
        scratch_types=[
            pltpu.VMEM((nch, CH), jnp.int32),
            pltpu.VMEM((nch, CH), jnp.int32),
            pltpu.VMEM((CH, F), jnp.float32),
            pltpu.VMEM_SHARED((np_, F), jnp.float32),
            pltpu.SemaphoreType.DMA,
        ],
        compiler_params=pltpu.CompilerParams(use_tc_tiling_on_sc=False),
    )
    def k(xs_h, src_h, dst_h, z_h, out_h, src_v, dst_v, rows_v, acc_s, sem):
        cid = lax.axis_index("c")
        sid = lax.axis_index("s")
        wid = cid * NSUB + sid
        pltpu.sync_copy(z_h, acc_s.at[pl.ds(sid * per, per)])
        pltpu.sync_copy(src_h.at[wid], src_v)
        pltpu.sync_copy(dst_h.at[wid], dst_v)
        plsc.subcore_barrier()

        def body(j, carry):
            pltpu.async_copy(xs_h.at[src_v.at[j]], rows_v, sem).wait()
            pltpu.sync_copy(rows_v, acc_s.at[dst_v.at[j]], add=True)
            return carry

        lax.fori_loop(0, nch, body, 0)
        plsc.subcore_barrier()
        off = pl.multiple_of(wid * per, per)
        pltpu.sync_copy(acc_s.at[pl.ds(sid * per, per)],
                        out_h.at[pl.ds(off, per)])

    return k(xs, src3, dst3, ztile)


def _tc_scale(degp, x24, np_):
    def body(deg_ref, x_ref, xs_ref, dinv_ref):
        deg = deg_ref[0] + deg_ref[1] + 1.0
        dinv = lax.rsqrt(deg)
        xs_ref[...] = x_ref[...] * dinv[:, None]
        dinv_ref[...] = dinv[:, None]

    grid = np_ // BN
    return pl.pallas_call(
        body,
        grid=(grid,),
        in_specs=[
            pl.BlockSpec((2, BN), lambda i: (0, i)),
            pl.BlockSpec((BN, F), lambda i: (i, 0)),
        ],
        out_specs=[
            pl.BlockSpec((BN, F), lambda i: (i, 0)),
            pl.BlockSpec((BN, 1), lambda i: (i, 0)),
        ],
        out_shape=[
            jax.ShapeDtypeStruct((np_, F), jnp.float32),
            jax.ShapeDtypeStruct((np_, 1), jnp.float32),
        ],
    )(degp, x24)


def _tc_gru(Spart, xs, dinv, W3, L3, b3, lb3, att, Wlin, blin, np_):
    def body(S_ref, xs_ref, dinv_ref, W3_ref, L3_ref, b3_ref, lb3_ref,
             att_ref, Wlin_ref, blin_ref, out_ref):
        ax = (S_ref[0] + S_ref[1] + xs_ref[...]) * dinv_ref[...]
        att_v = att_ref[...]
        e = jnp.exp(att_v - jnp.max(att_v))
        probs = e / jnp.sum(e)

        f32 = jnp.float32
        WzL = jnp.dot(W3_ref[0], L3_ref[0, :HID, :], preferred_element_type=f32)
        WrL = jnp.dot(W3_ref[1], L3_ref[1, :HID, :], preferred_element_type=f32)
        WhL = jnp.dot(W3_ref[2], L3_ref[2, :HID, :], preferred_element_type=f32)
        cz = jnp.dot(b3_ref[0:1, :], L3_ref[0, :HID, :],
                     preferred_element_type=f32) + lb3_ref[0:1, :]
        cr = jnp.dot(b3_ref[1:2, :], L3_ref[1, :HID, :],
                     preferred_element_type=f32) + lb3_ref[1:2, :]
        chh = jnp.dot(b3_ref[2:3, :], L3_ref[2, :HID, :],
                      preferred_element_type=f32) + lb3_ref[2:3, :]
        Lzr_bot = jnp.concatenate(
            [L3_ref[0, HID:, :], L3_ref[1, HID:, :]], axis=1)
        Lh_bot = L3_ref[2, HID:, :]

        Hs = jnp.zeros((BN, HID), f32)
        Hacc = jnp.zeros((BN, HID), f32)
        for p in range(P):
            x0 = ax[:, p:p + 1]
            x1 = ax[:, P + p:P + p + 1]
            zin = x0 * WzL[0:1, :] + x1 * WzL[1:2, :] + cz
            rin = x0 * WrL[0:1, :] + x1 * WrL[1:2, :] + cr
            hin = x0 * WhL[0:1, :] + x1 * WhL[1:2, :] + chh
            zr = jnp.dot(Hs, Lzr_bot, preferred_element_type=f32)
            Z = jax.nn.sigmoid(zr[:, :HID] + zin)
            R = jax.nn.sigmoid(zr[:, HID:] + rin)
            Ht = jnp.tanh(jnp.dot(Hs * R, Lh_bot,
                                  preferred_element_type=f32) + hin)
            Hs = Z * Hs + (1.0 - Z) * Ht
            Hacc = Hacc + probs[:, p:p + 1] * Hs
        out_ref[...] = (jnp.dot(jnp.maximum(Hacc, 0.0), Wlin_ref[...],
                                preferred_element_type=f32)
                        + blin_ref[...])

    grid = np_ // BN
    fixed = lambda i: (0, 0)
    fixed3 = lambda i: (0, 0, 0)
    return pl.pallas_call(
        body,
        grid=(grid,),
        in_specs=[
            pl.BlockSpec((2, BN, F), lambda i: (0, i, 0)),
            pl.BlockSpec((BN, F), lambda i: (i, 0)),
            pl.BlockSpec((BN, 1), lambda i: (i, 0)),
            pl.BlockSpec((3, 2, HID), fixed3),
            pl.BlockSpec((3, 2 * HID, HID), fixed3),
            pl.BlockSpec((3, HID), fixed),
            pl.BlockSpec((3, HID), fixed),
            pl.BlockSpec((1, P), fixed),
            pl.BlockSpec((HID, P), fixed),
            pl.BlockSpec((1, P), fixed),
        ],
        out_specs=pl.BlockSpec((BN, P), lambda i: (i, 0)),
        out_shape=jax.ShapeDtypeStruct((np_, P), jnp.float32),
    )(Spart, xs, dinv, W3, L3, b3, lb3, att, Wlin, blin)


def kernel(x, edge_index, Wz, bz, Lz, lbz, Wr, br, Lr, lbr, Wh, bh, Lh, lbh,
           att, Wlin, blin):
    n = x.shape[0]
    e = edge_index.shape[1]

    np_ = -(-n // 2048) * 2048
    dustbin = np_ - 1

    nch = -(-e // (NCORE * NSUB * CH))
    pe = NCORE * NSUB * nch * CH
    pad = pe - e
    src = jnp.concatenate(
        [edge_index[0], jnp.full((pad,), dustbin, jnp.int32)])
    dst = jnp.concatenate(
        [edge_index[1], jnp.full((pad,), dustbin, jnp.int32)])
    src3 = src.reshape(NCORE * NSUB, nch, CH)
    dst3 = dst.reshape(NCORE * NSUB, nch, CH)

    per = np_ // NSUB
    zrow = jnp.zeros((per,), jnp.float32)
    ztile = jnp.zeros((per, F), jnp.float32)

    x24 = x.reshape(n, F)
    x24p = jnp.pad(x24, ((0, np_ - n), (0, 0)))

    degp = _sc_degree(dst3, zrow, np_, nch).reshape(NCORE, np_)
    xs, dinv = _tc_scale(degp, x24p, np_)
    Spart = _sc_scatter(xs, src3, dst3, ztile, np_, nch).reshape(NCORE, np_, F)

    W3 = jnp.stack([Wz, Wr, Wh])
    L3 = jnp.stack([Lz, Lr, Lh])
    b3 = jnp.stack([bz, br, bh])
    lb3 = jnp.stack([lbz, lbr, lbh])
    out = _tc_gru(Spart, xs, dinv, W3, L3, b3, lb3,
                  att.reshape(1, P), Wlin, blin.reshape(1, P), np_)
    return out[:n]


# speedup vs baseline: 336.7842x; 2.0668x over previous
import functools

import jax
import jax.numpy as jnp
from jax import lax
from jax.experimental import pallas as pl
from jax.experimental.pallas import tpu as pltpu
from jax.experimental.pallas import tpu_sc as plsc

NCORE = 2
NSUB = 16
CH = 128
F = 24
HID = 32
P = 12
BN = 512


def _sc_degree(dst3, zrow, np_, nch):
    per = np_ // NSUB
    mesh = plsc.VectorSubcoreMesh(core_axis_name="c", subcore_axis_name="s")

    @functools.partial(
        pl.kernel,
        out_type=jax.ShapeDtypeStruct((NCORE * np_,), jnp.float32),
        mesh=mesh,
        scratch_types=[
            pltpu.VMEM((nch, CH), jnp.int32),
            pltpu.VMEM((CH,), jnp.float32),
            pltpu.VMEM_SHARED((np_,), jnp.float32),
        ],
    )
    def k(dst_h, z_h, out_h, idx_v, ones_v, deg_s):
        cid = lax.axis_index("c")
        sid = lax.axis_index("s")
        wid = cid * NSUB + sid
        pltpu.sync_copy(z_h, deg_s.at[pl.ds(sid * per, per)])
        pltpu.sync_copy(dst_h.at[wid], idx_v)
        for i in range(CH // 16):
            ones_v[pl.ds(i * 16, 16)] = jnp.ones((16,), jnp.float32)
        plsc.subcore_barrier()

        def body(j, carry):
            pltpu.sync_copy(ones_v, deg_s.at[idx_v.at[j]], add=True)
            return carry

        lax.fori_loop(0, nch, body, 0)
        plsc.subcore_barrier()
        off = pl.multiple_of(wid * per, per)
        pltpu.sync_copy(deg_s.at[pl.ds(sid * per, per)],
                        out_h.at[pl.ds(off, per)])

    return k(dst3, zrow)


def _sc_scatter(xs, src4, dst4, ztile, np_, nblk, nb):
    per = np_ // NSUB
    mesh = plsc.VectorSubcoreMesh(core_axis_name="c", subcore_axis_name="s")

    @functools.partial(
        pl.kernel,
        out_type=jax.ShapeDtypeStruct((NCORE * np_, F), jnp.float32),
        mesh=mesh,
        scratch_types=[
            pltpu.VMEM((nb, CH), jnp.int32),
            pltpu.VMEM((nb, CH), jnp.int32),
            pltpu.VMEM((2, CH, F), jnp.float32),
            pltpu.VMEM_SHARED((np_, F), jnp.float32),
            pltpu.SemaphoreType.DMA,
            pltpu.SemaphoreType.DMA,
        ],
        compiler_params=pltpu.CompilerParams(use_tc_tiling_on_sc=False),
    )
    def k(xs_h, src_h, dst_h, z_h, out_h, src_v, dst_v, rows_v, acc_s,
          sem0, sem1):
        cid = lax.axis_index("c")
        sid = lax.axis_index("s")
        wid = cid * NSUB + sid
        pltpu.sync_copy(z_h, acc_s.at[pl.ds(sid * per, per)])
        plsc.subcore_barrier()

        sems = (sem0, sem1)

        @pl.loop(0, nblk)
        def block(t):
            pltpu.sync_copy(src_h.at[wid, t], src_v)
            pltpu.sync_copy(dst_h.at[wid, t], dst_v)
            pltpu.async_copy(xs_h.at[src_v.at[0]], rows_v.at[0], sem0)
            for c in range(nb):
                b = c % 2
                if c + 1 < nb:
                    pltpu.async_copy(xs_h.at[src_v.at[c + 1]],
                                     rows_v.at[1 - b], sems[1 - b])
                pltpu.make_async_copy(xs_h.at[src_v.at[c]],
                                      rows_v.at[b], sems[b]).wait()
                pltpu.sync_copy(rows_v.at[b], acc_s.at[dst_v.at[c]],
                                add=True)

        plsc.subcore_barrier()
        off = pl.multiple_of(wid * per, per)
        pltpu.sync_copy(acc_s.at[pl.ds(sid * per, per)],
                        out_h.at[pl.ds(off, per)])

    return k(xs, src4, dst4, ztile)


def _tc_scale(degp, x24, np_):
    def body(deg_ref, x_ref, xs_ref, dinv_ref):
        deg = deg_ref[0] + deg_ref[1] + 1.0
        dinv = lax.rsqrt(deg)
        xs_ref[...] = x_ref[...] * dinv[:, None]
        dinv_ref[...] = dinv[:, None]

    grid = np_ // BN
    return pl.pallas_call(
        body,
        grid=(grid,),
        in_specs=[
            pl.BlockSpec((2, BN), lambda i: (0, i)),
            pl.BlockSpec((BN, F), lambda i: (i, 0)),
        ],
        out_specs=[
            pl.BlockSpec((BN, F), lambda i: (i, 0)),
            pl.BlockSpec((BN, 1), lambda i: (i, 0)),
        ],
        out_shape=[
            jax.ShapeDtypeStruct((np_, F), jnp.float32),
            jax.ShapeDtypeStruct((np_, 1), jnp.float32),
        ],
    )(degp, x24)


def _tc_gru(Spart, xs, dinv, W3, L3, b3, lb3, att, Wlin, blin, np_):
    def body(S_ref, xs_ref, dinv_ref, W3_ref, L3_ref, b3_ref, lb3_ref,
             att_ref, Wlin_ref, blin_ref, out_ref):
        ax = (S_ref[0] + S_ref[1] + xs_ref[...]) * dinv_ref[...]
        att_v = att_ref[...]
        e = jnp.exp(att_v - jnp.max(att_v))
        probs = e / jnp.sum(e)

        f32 = jnp.float32
        WzL = jnp.dot(W3_ref[0], L3_ref[0, :HID, :], preferred_element_type=f32)
        WrL = jnp.dot(W3_ref[1], L3_ref[1, :HID, :], preferred_element_type=f32)
        WhL = jnp.dot(W3_ref[2], L3_ref[2, :HID, :], preferred_element_type=f32)
        cz = jnp.dot(b3_ref[0:1, :], L3_ref[0, :HID, :],
                     preferred_element_type=f32) + lb3_ref[0:1, :]
        cr = jnp.dot(b3_ref[1:2, :], L3_ref[1, :HID, :],
                     preferred_element_type=f32) + lb3_ref[1:2, :]
        chh = jnp.dot(b3_ref[2:3, :], L3_ref[2, :HID, :],
                      preferred_element_type=f32) + lb3_ref[2:3, :]
        Lzr_bot = jnp.concatenate(
            [L3_ref[0, HID:, :], L3_ref[1, HID:, :]], axis=1)
        Lh_bot = L3_ref[2, HID:, :]

        Hs = jnp.zeros((BN, HID), f32)
        Hacc = jnp.zeros((BN, HID), f32)
        for p in range(P):
            x0 = ax[:, p:p + 1]
            x1 = ax[:, P + p:P + p + 1]
            zin = x0 * WzL[0:1, :] + x1 * WzL[1:2, :] + cz
            rin = x0 * WrL[0:1, :] + x1 * WrL[1:2, :] + cr
            hin = x0 * WhL[0:1, :] + x1 * WhL[1:2, :] + chh
            zr = jnp.dot(Hs, Lzr_bot, preferred_element_type=f32)
            Z = jax.nn.sigmoid(zr[:, :HID] + zin)
            R = jax.nn.sigmoid(zr[:, HID:] + rin)
            Ht = jnp.tanh(jnp.dot(Hs * R, Lh_bot,
                                  preferred_element_type=f32) + hin)
            Hs = Z * Hs + (1.0 - Z) * Ht
            Hacc = Hacc + probs[:, p:p + 1] * Hs
        out_ref[...] = (jnp.dot(jnp.maximum(Hacc, 0.0), Wlin_ref[...],
                                preferred_element_type=f32)
                        + blin_ref[...])

    grid = np_ // BN
    fixed = lambda i: (0, 0)
    fixed3 = lambda i: (0, 0, 0)
    return pl.pallas_call(
        body,
        grid=(grid,),
        in_specs=[
            pl.BlockSpec((2, BN, F), lambda i: (0, i, 0)),
            pl.BlockSpec((BN, F), lambda i: (i, 0)),
            pl.BlockSpec((BN, 1), lambda i: (i, 0)),
            pl.BlockSpec((3, 2, HID), fixed3),
            pl.BlockSpec((3, 2 * HID, HID), fixed3),
            pl.BlockSpec((3, HID), fixed),
            pl.BlockSpec((3, HID), fixed),
            pl.BlockSpec((1, P), fixed),
            pl.BlockSpec((HID, P), fixed),
            pl.BlockSpec((1, P), fixed),
        ],
        out_specs=pl.BlockSpec((BN, P), lambda i: (i, 0)),
        out_shape=jax.ShapeDtypeStruct((np_, P), jnp.float32),
    )(Spart, xs, dinv, W3, L3, b3, lb3, att, Wlin, blin)


def kernel(x, edge_index, Wz, bz, Lz, lbz, Wr, br, Lr, lbr, Wh, bh, Lh, lbh,
           att, Wlin, blin):
    n = x.shape[0]
    e = edge_index.shape[1]

    np_ = -(-n // 2048) * 2048
    dustbin = np_ - 1

    NB = 28
    nch = -(-e // (NCORE * NSUB * CH))
    nch = -(-nch // NB) * NB
    nblk = nch // NB
    pe = NCORE * NSUB * nch * CH
    pad = pe - e
    src = jnp.concatenate(
        [edge_index[0], jnp.full((pad,), dustbin, jnp.int32)])
    dst = jnp.concatenate(
        [edge_index[1], jnp.full((pad,), dustbin, jnp.int32)])
    dst3 = dst.reshape(NCORE * NSUB, nch, CH)
    src4 = src.reshape(NCORE * NSUB, nblk, NB, CH)
    dst4 = dst.reshape(NCORE * NSUB, nblk, NB, CH)

    per = np_ // NSUB
    zrow = jnp.zeros((per,), jnp.float32)
    ztile = jnp.zeros((per, F), jnp.float32)

    x24 = x.reshape(n, F)
    x24p = jnp.pad(x24, ((0, np_ - n), (0, 0)))

    degp = _sc_degree(dst3, zrow, np_, nch).reshape(NCORE, np_)
    xs, dinv = _tc_scale(degp, x24p, np_)
    Spart = _sc_scatter(xs, src4, dst4, ztile, np_, nblk,
                        NB).reshape(NCORE, np_, F)

    W3 = jnp.stack([Wz, Wr, Wh])
    L3 = jnp.stack([Lz, Lr, Lh])
    b3 = jnp.stack([bz, br, bh])
    lb3 = jnp.stack([lbz, lbr, lbh])
    out = _tc_gru(Spart, xs, dinv, W3, L3, b3, lb3,
                  att.reshape(1, P), Wlin, blin.reshape(1, P), np_)
    return out[:n]
